# Initial kernel scaffold; baseline (speedup 1.0000x reference)
import jax
import jax.numpy as jnp
from jax.experimental import pallas as pl


def kernel(matches, rating):
    raise NotImplementedError("write your pallas kernel here")


# speedup vs baseline: 1.7223x; 1.7223x over previous
import math

import jax
import jax.numpy as jnp
import numpy as np
from jax import lax
from jax.experimental import pallas as pl
from jax.experimental.pallas import tpu as pltpu
from jax.experimental.pallas import tpu_sc as plsc

_C = 3.0
_D = 500.0
_BATCH = 16384
_NC = 2
_NS = 16
_NW = _NC * _NS
_BPW = _BATCH // _NW
_LANES = 16
_ROWS = _BPW // 128
_K = np.float32(math.log(_C) / _D)


def _elo_body(mh_hbm, ma_hbm, tab_hbm, out_hbm, idx_h, idx_a, rows_h, rows_a, out_v, sem):
    cid = lax.axis_index("c")
    sid = lax.axis_index("s")
    wid = sid * _NC + cid

    pltpu.sync_copy(mh_hbm.at[wid], idx_h)
    pltpu.sync_copy(ma_hbm.at[wid], idx_a)

    cps = []
    for j in range(_ROWS):
        j32 = np.int32(j)
        cps.append(pltpu.async_copy(tab_hbm.at[idx_h.at[j32]], rows_h.at[j32], sem))
        cps.append(pltpu.async_copy(tab_hbm.at[idx_a.at[j32]], rows_a.at[j32], sem))
    for cp in cps:
        cp.wait()

    one = np.float32(1.0)
    for j in range(_ROWS):
        j32 = np.int32(j)
        for c in range(128 // _LANES):
            sl = pl.ds(c * _LANES, _LANES)
            r_h = rows_h[j32, sl]
            r_a = rows_a[j32, sl]
            x = (r_a - r_h) * _K
            out_v[j32, sl] = one / (one + jnp.exp(x))

    pltpu.sync_copy(out_v, out_hbm.at[wid])


def kernel(matches, rating):
    mi = matches.astype(jnp.int32).reshape(2, _NW, _ROWS, 128)
    tab = rating.astype(jnp.float32)
    mh, ma = mi[0], mi[1]
    mesh = plsc.VectorSubcoreMesh(core_axis_name="c", subcore_axis_name="s")
    out = pl.kernel(
        _elo_body,
        mesh=mesh,
        out_type=jax.ShapeDtypeStruct((_NW, _ROWS, 128), jnp.float32),
        scratch_types=[
            pltpu.VMEM((_ROWS, 128), jnp.int32),
            pltpu.VMEM((_ROWS, 128), jnp.int32),
            pltpu.VMEM((_ROWS, 128), jnp.float32),
            pltpu.VMEM((_ROWS, 128), jnp.float32),
            pltpu.VMEM((_ROWS, 128), jnp.float32),
            pltpu.SemaphoreType.DMA,
        ],
    )(mh, ma, tab)
    return out.reshape(_BATCH).astype(jnp.float64)
